# Initial kernel scaffold; baseline (speedup 1.0000x reference)
import functools

import jax
import jax.numpy as jnp
from jax import lax
from jax.experimental import pallas as pl
from jax.experimental.pallas import tpu as pltpu
from jax.experimental.pallas import tpu_sc as plsc

NC = 2
NS = 16
CH = 128


def _make_seg_kernel(n_acc, n_src_rows, e_pad, d, with_counts):
  rpt = n_acc // NS
  ept = e_pad // NS
  nch = ept // CH

  out_type = [jax.ShapeDtypeStruct((n_acc, d), jnp.float32)] * 2
  scratch = [
      pltpu.VMEM_SHARED((n_acc, d), jnp.float32),
      pltpu.VMEM((CH,), jnp.int32),
      pltpu.VMEM((CH,), jnp.int32),
      pltpu.VMEM((CH, d), jnp.float32),
  ]
  if with_counts:
    out_type += [jax.ShapeDtypeStruct((n_acc, 16), jnp.float32)] * 2
    scratch += [
        pltpu.VMEM_SHARED((n_acc, 16), jnp.float32),
        pltpu.VMEM((CH, 16), jnp.float32),
    ]

  mesh = plsc.VectorSubcoreMesh(core_axis_name="c", subcore_axis_name="s",
                                num_cores=NC, num_subcores=NS)

  def body(*refs):
    if with_counts:
      (x0, x1, s0, d0, s1, d1, z_d, z_c, ones_h,
       sum0, sum1, cnt0, cnt1, acc, sidx, didx, rows, cntacc, ones_v) = refs
    else:
      (x0, x1, s0, d0, s1, d1, z_d,
       sum0, sum1, acc, sidx, didx, rows) = refs
    cid = lax.axis_index("c")
    sid = lax.axis_index("s")
    r0 = sid * rpt

    pltpu.sync_copy(z_d.at[pl.ds(r0, rpt)], acc.at[pl.ds(r0, rpt)])
    if with_counts:
      pltpu.sync_copy(z_c.at[pl.ds(r0, rpt)], cntacc.at[pl.ds(r0, rpt)])
      pltpu.sync_copy(ones_h, ones_v)
    plsc.subcore_barrier()

    def do_edges(x_hbm, s_hbm, d_hbm):
      def step(i, carry):
        b = sid * ept + i * CH
        pltpu.sync_copy(s_hbm.at[pl.ds(b, CH)], sidx)
        pltpu.sync_copy(d_hbm.at[pl.ds(b, CH)], didx)
        pltpu.sync_copy(x_hbm.at[sidx], rows)
        pltpu.sync_copy(rows, acc.at[didx], add=True)
        if with_counts:
          pltpu.sync_copy(ones_v, cntacc.at[didx], add=True)
        return carry
      lax.fori_loop(0, nch, step, 0)

    pl.when(cid == 0)(lambda: do_edges(x0, s0, d0))
    pl.when(cid == 1)(lambda: do_edges(x1, s1, d1))
    plsc.subcore_barrier()

    def writeout(o_sum, o_cnt):
      pltpu.sync_copy(acc.at[pl.ds(r0, rpt)], o_sum.at[pl.ds(r0, rpt)])
      if o_cnt is not None:
        pltpu.sync_copy(cntacc.at[pl.ds(r0, rpt)], o_cnt.at[pl.ds(r0, rpt)])

    if with_counts:
      pl.when(cid == 0)(lambda: writeout(sum0, cnt0))
      pl.when(cid == 1)(lambda: writeout(sum1, cnt1))
    else:
      pl.when(cid == 0)(lambda: writeout(sum0, None))
      pl.when(cid == 1)(lambda: writeout(sum1, None))

  return pl.kernel(body, out_type=out_type, mesh=mesh, scratch_types=scratch)


def _make_dense_kernel(n, n_acc, d, out_rows):
  def one(s_ref, c_ref, x_ref, wl_ref, bl_ref, wr_ref, g_ref, be_ref, o_ref):
    cnt = jnp.maximum(c_ref[0:n, 0:1], 1.0)
    mean = s_ref[0:n, :] / cnt
    h = jnp.dot(mean, wl_ref[...], preferred_element_type=jnp.float32)
    h = h + bl_ref[...]
    h = h + jnp.dot(x_ref[...], wr_ref[...], preferred_element_type=jnp.float32)
    mu = jnp.mean(h, axis=0, keepdims=True)
    var = jnp.mean(jnp.square(h - mu), axis=0, keepdims=True)
    y = (h - mu) * lax.rsqrt(var + 1e-5) * g_ref[...] + be_ref[...]
    y = jnp.where(y > 0, y, jnp.exp(jnp.minimum(y, 0.0)) - 1.0)
    o_ref[0:n, :] = y
    if out_rows > n:
      o_ref[n:out_rows, :] = jnp.zeros((out_rows - n, d), jnp.float32)

  def body(s0, c0, x0, wl0, bl0, wr0, g0, be0,
           s1, c1, x1, wl1, bl1, wr1, g1, be1, o0, o1):
    one(s0, c0, x0, wl0, bl0, wr0, g0, be0, o0)
    one(s1, c1, x1, wl1, bl1, wr1, g1, be1, o1)

  return pl.pallas_call(
      body,
      out_shape=[jax.ShapeDtypeStruct((out_rows, d), jnp.float32)] * 2,
  )


def kernel(x_user, x_item, edge_index_ui, edge_index_iu,
           Wl0_ui, bl0_ui, Wr0_ui, Wl0_iu, bl0_iu, Wr0_iu,
           g0_u, be0_u, g0_i, be0_i,
           Wl1_ui, bl1_ui, Wr1_ui, Wl1_iu, bl1_iu, Wr1_iu,
           g1_u, be1_u, g1_i, be1_i):
  n, d = x_user.shape
  e = edge_index_ui.shape[1]

  n_acc = ((n + NS) // NS) * NS
  n_src = n + 8
  e_pad = -(-e // (NS * CH)) * (NS * CH)

  i32 = jnp.int32
  pad_idx = jnp.full((e_pad - e,), n, i32)
  s_ui = jnp.concatenate([edge_index_ui[0].astype(i32), pad_idx])
  d_ui = jnp.concatenate([edge_index_ui[1].astype(i32), pad_idx])
  s_iu = jnp.concatenate([edge_index_iu[0].astype(i32), pad_idx])
  d_iu = jnp.concatenate([edge_index_iu[1].astype(i32), pad_idx])

  zrow = jnp.zeros((n_src - n, d), jnp.float32)
  xu_pad = jnp.concatenate([x_user, zrow])
  xi_pad = jnp.concatenate([x_item, zrow])
  z_d = jnp.zeros((n_acc, d), jnp.float32)
  z_c = jnp.zeros((n_acc, 16), jnp.float32)
  ones_h = jnp.ones((CH, 16), jnp.float32)

  seg_c = _make_seg_kernel(n_acc, n_src, e_pad, d, with_counts=True)
  seg_n = _make_seg_kernel(n_acc, n_src, e_pad, d, with_counts=False)
  dense_pad = _make_dense_kernel(n, n_acc, d, n_src)
  dense_fin = _make_dense_kernel(n, n_acc, d, n)

  r2 = lambda v: v.reshape(1, d)

  sum_i0, sum_u0, cnt_i, cnt_u = seg_c(
      xu_pad, xi_pad, s_ui, d_ui, s_iu, d_iu, z_d, z_c, ones_h)
  i1_pad, u1_pad = dense_pad(
      sum_i0, cnt_i, x_item, Wl0_ui, r2(bl0_ui), Wr0_ui, r2(g0_i), r2(be0_i),
      sum_u0, cnt_u, x_user, Wl0_iu, r2(bl0_iu), Wr0_iu, r2(g0_u), r2(be0_u))

  sum_i1, sum_u1 = seg_n(u1_pad, i1_pad, s_ui, d_ui, s_iu, d_iu, z_d)
  i2, u2 = dense_fin(
      sum_i1, cnt_i, i1_pad[0:n], Wl1_ui, r2(bl1_ui), Wr1_ui,
      r2(g1_i), r2(be1_i),
      sum_u1, cnt_u, u1_pad[0:n], Wl1_iu, r2(bl1_iu), Wr1_iu,
      r2(g1_u), r2(be1_u))

  return (x_user, x_item, u1_pad[0:n], i1_pad[0:n], u2, i2)


# speedup vs baseline: 3.4659x; 3.4659x over previous
import functools

import jax
import jax.numpy as jnp
from jax import lax
from jax.experimental import pallas as pl
from jax.experimental.pallas import tpu as pltpu
from jax.experimental.pallas import tpu_sc as plsc

NC = 2
NS = 16
CH = 128


def _make_seg_kernel(n_acc, n_src_rows, e_pad, d, with_counts):
  rpt = n_acc // NS
  ept = e_pad // NS
  nch = ept // CH

  out_type = [jax.ShapeDtypeStruct((n_acc, d), jnp.float32)] * (
      4 if with_counts else 2)
  scratch = [
      pltpu.VMEM_SHARED((n_acc, d), jnp.float32),
      pltpu.VMEM((CH,), jnp.int32),
      pltpu.VMEM((CH,), jnp.int32),
      pltpu.VMEM((CH, d), jnp.float32),
  ]

  mesh = plsc.VectorSubcoreMesh(core_axis_name="c", subcore_axis_name="s",
                                num_cores=NC, num_subcores=NS)

  def body(*refs):
    if with_counts:
      (x0, x1, s0, d0, s1, d1, z_d,
       sum0, sum1, cnt0, cnt1, acc, sidx, didx, rows) = refs
    else:
      (x0, x1, s0, d0, s1, d1, z_d,
       sum0, sum1, acc, sidx, didx, rows) = refs
    cid = lax.axis_index("c")
    sid = lax.axis_index("s")
    r0 = sid * rpt

    def zero_acc():
      for j in range(rpt // CH):
        pltpu.sync_copy(z_d.at[pl.ds(r0 + j * CH, CH)], rows)
        pltpu.sync_copy(rows, acc.at[pl.ds(r0 + j * CH, CH)])

    def writeout(o_ref):
      for j in range(rpt // CH):
        pltpu.sync_copy(acc.at[pl.ds(r0 + j * CH, CH)], rows)
        pltpu.sync_copy(rows, o_ref.at[pl.ds(r0 + j * CH, CH)])

    zero_acc()
    plsc.subcore_barrier()

    def do_edges(x_hbm, s_hbm, d_hbm):
      def step(i, carry):
        b = sid * ept + i * CH
        pltpu.sync_copy(s_hbm.at[pl.ds(b, CH)], sidx)
        pltpu.sync_copy(d_hbm.at[pl.ds(b, CH)], didx)
        pltpu.sync_copy(x_hbm.at[sidx], rows)
        pltpu.sync_copy(rows, acc.at[didx], add=True)
        return carry
      lax.fori_loop(0, nch, step, 0)

    pl.when(cid == 0)(lambda: do_edges(x0, s0, d0))
    pl.when(cid == 1)(lambda: do_edges(x1, s1, d1))
    plsc.subcore_barrier()
    pl.when(cid == 0)(lambda: writeout(sum0))
    pl.when(cid == 1)(lambda: writeout(sum1))

    if with_counts:
      plsc.subcore_barrier()
      zero_acc()
      ones16 = jnp.ones((16,), jnp.float32)

      def fill_ones(r, carry):
        def fill_cols(c, carry2):
          rows[r, pl.ds(c * 16, 16)] = ones16
          return carry2
        return lax.fori_loop(0, d // 16, fill_cols, carry)
      lax.fori_loop(0, CH, fill_ones, 0)
      plsc.subcore_barrier()

      def do_counts(d_hbm):
        def step(i, carry):
          b = sid * ept + i * CH
          pltpu.sync_copy(d_hbm.at[pl.ds(b, CH)], didx)
          pltpu.sync_copy(rows, acc.at[didx], add=True)
          return carry
        lax.fori_loop(0, nch, step, 0)

      pl.when(cid == 0)(lambda: do_counts(d0))
      pl.when(cid == 1)(lambda: do_counts(d1))
      plsc.subcore_barrier()
      pl.when(cid == 0)(lambda: writeout(cnt0))
      pl.when(cid == 1)(lambda: writeout(cnt1))

  return pl.kernel(body, out_type=out_type, mesh=mesh, scratch_types=scratch)


def _make_dense_kernel(n, n_acc, d, out_rows):
  def one(s_ref, c_ref, x_ref, wl_ref, bl_ref, wr_ref, g_ref, be_ref, o_ref):
    cnt = jnp.maximum(c_ref[0:n, 0:1], 1.0)
    mean = s_ref[0:n, :] / cnt
    h = jnp.dot(mean, wl_ref[...], preferred_element_type=jnp.float32)
    h = h + bl_ref[...]
    h = h + jnp.dot(x_ref[...], wr_ref[...], preferred_element_type=jnp.float32)
    mu = jnp.mean(h, axis=0, keepdims=True)
    var = jnp.mean(jnp.square(h - mu), axis=0, keepdims=True)
    y = (h - mu) * lax.rsqrt(var + 1e-5) * g_ref[...] + be_ref[...]
    y = jnp.where(y > 0, y, jnp.exp(jnp.minimum(y, 0.0)) - 1.0)
    o_ref[0:n, :] = y
    if out_rows > n:
      o_ref[n:out_rows, :] = jnp.zeros((out_rows - n, d), jnp.float32)

  def body(s0, c0, x0, wl0, bl0, wr0, g0, be0,
           s1, c1, x1, wl1, bl1, wr1, g1, be1, o0, o1):
    one(s0, c0, x0, wl0, bl0, wr0, g0, be0, o0)
    one(s1, c1, x1, wl1, bl1, wr1, g1, be1, o1)

  return pl.pallas_call(
      body,
      out_shape=[jax.ShapeDtypeStruct((out_rows, d), jnp.float32)] * 2,
  )


def kernel(x_user, x_item, edge_index_ui, edge_index_iu,
           Wl0_ui, bl0_ui, Wr0_ui, Wl0_iu, bl0_iu, Wr0_iu,
           g0_u, be0_u, g0_i, be0_i,
           Wl1_ui, bl1_ui, Wr1_ui, Wl1_iu, bl1_iu, Wr1_iu,
           g1_u, be1_u, g1_i, be1_i):
  n, d = x_user.shape
  e = edge_index_ui.shape[1]

  n_acc = -(-(n + 1) // (NS * CH)) * (NS * CH)
  n_src = n + 8
  e_pad = -(-e // (NS * CH)) * (NS * CH)

  i32 = jnp.int32
  pad_idx = jnp.full((e_pad - e,), n, i32)
  s_ui = jnp.concatenate([edge_index_ui[0].astype(i32), pad_idx])
  d_ui = jnp.concatenate([edge_index_ui[1].astype(i32), pad_idx])
  s_iu = jnp.concatenate([edge_index_iu[0].astype(i32), pad_idx])
  d_iu = jnp.concatenate([edge_index_iu[1].astype(i32), pad_idx])

  zrow = jnp.zeros((n_src - n, d), jnp.float32)
  xu_pad = jnp.concatenate([x_user, zrow])
  xi_pad = jnp.concatenate([x_item, zrow])
  z_d = jnp.zeros((n_acc, d), jnp.float32)

  seg_c = _make_seg_kernel(n_acc, n_src, e_pad, d, with_counts=True)
  seg_n = _make_seg_kernel(n_acc, n_src, e_pad, d, with_counts=False)
  dense_pad = _make_dense_kernel(n, n_acc, d, n_src)
  dense_fin = _make_dense_kernel(n, n_acc, d, n)

  r2 = lambda v: v.reshape(1, d)

  sum_i0, sum_u0, cnt_i, cnt_u = seg_c(
      xu_pad, xi_pad, s_ui, d_ui, s_iu, d_iu, z_d)
  i1_pad, u1_pad = dense_pad(
      sum_i0, cnt_i, x_item, Wl0_ui, r2(bl0_ui), Wr0_ui, r2(g0_i), r2(be0_i),
      sum_u0, cnt_u, x_user, Wl0_iu, r2(bl0_iu), Wr0_iu, r2(g0_u), r2(be0_u))

  sum_i1, sum_u1 = seg_n(u1_pad, i1_pad, s_ui, d_ui, s_iu, d_iu, z_d)
  i2, u2 = dense_fin(
      sum_i1, cnt_i, i1_pad[0:n], Wl1_ui, r2(bl1_ui), Wr1_ui,
      r2(g1_i), r2(be1_i),
      sum_u1, cnt_u, u1_pad[0:n], Wl1_iu, r2(bl1_iu), Wr1_iu,
      r2(g1_u), r2(be1_u))

  return (x_user, x_item, u1_pad[0:n], i1_pad[0:n], u2, i2)
